# baseline (speedup 1.0000x reference)
import jax
import jax.numpy as jnp
from jax import lax
from jax.experimental import pallas as pl
from jax.experimental.pallas import tpu as pltpu
from jax.experimental.pallas import tpu_sc as plsc

F = 26
V = 100000
D = 16
NUM = 13
B = 4096
NC = 2
NS = 16
NW = NC * NS
RPW = B // NW
GPW = RPW * F
SPR = 32
GPE = RPW * SPR
CHUNK = 128
NCHUNK = GPW // CHUNK
NCHE = GPE // CHUNK


def _sc_body(emb_hbm, fm_hbm, gidx_hbm, idx_hbm, rows_out, fm_out,
             gidx_v, idx_v, rows_v, fmv, fm_res, sem_e, sem_f, sem_o):
    wid = lax.axis_index("s") * NC + lax.axis_index("c")
    base = wid * GPW

    pltpu.sync_copy(gidx_hbm.at[pl.ds(wid * GPE, GPE)], gidx_v)
    pltpu.sync_copy(idx_hbm.at[pl.ds(base, GPW)], idx_v)

    def fire(j, c):
        gsl = gidx_v.at[pl.ds(j * CHUNK, CHUNK)]
        pltpu.async_copy(emb_hbm.at[gsl], rows_v.at[pl.ds(j * CHUNK, CHUNK)], sem_e)

        @pl.when(j < NCHUNK)
        def _():
            isl = idx_v.at[pl.ds(j * CHUNK, CHUNK)]
            pltpu.async_copy(fm_hbm.at[isl], fmv.at[pl.ds(j * CHUNK, CHUNK)], sem_f)

        return c

    lax.fori_loop(0, NCHE, fire, 0)
    pltpu.make_async_copy(emb_hbm.at[pl.ds(0, GPE)], rows_v, sem_e).wait()
    pltpu.make_async_copy(fm_hbm.at[pl.ds(0, GPW)], fmv.at[pl.ds(0, GPW)], sem_f).wait()

    out_copy = pltpu.make_async_copy(rows_v, rows_out.at[pl.ds(wid * GPE, GPE)], sem_o)
    out_copy.start()

    lanes = lax.iota(jnp.int32, 16)
    tail_mask = lanes < (F - 16)
    rots = [((lanes + k) & 15)[:, None] for k in (1, 2, 4, 8)]
    dnums = lax.GatherDimensionNumbers(
        offset_dims=(), collapsed_slice_dims=(0,), start_index_map=(0,))

    def lane_sum(x):
        for r in rots:
            x = x + lax.gather(x, r, dimension_numbers=dnums, slice_sizes=(1,),
                               mode=lax.GatherScatterMode.PROMISE_IN_BOUNDS)
        return x

    def per_row(i, c):
        base_f = i * F
        v0 = fmv[pl.ds(base_f, 16)]
        v1 = jnp.where(tail_mask, fmv[pl.ds(base_f + 16, 16)], 0.0)
        fm_res[i] = lane_sum(v0 + v1)
        return c

    lax.fori_loop(0, RPW, per_row, 0)

    pltpu.sync_copy(fm_res, fm_out.at[pl.ds(wid * RPW, RPW)])
    out_copy.wait()


def _sc_gather_fm(emb_flat, fm_flat, gidx_flat, idx_flat):
    mesh = plsc.VectorSubcoreMesh(core_axis_name="c", subcore_axis_name="s")
    k = pl.kernel(
        _sc_body,
        out_type=(
            jax.ShapeDtypeStruct((B * SPR, D), jnp.bfloat16),
            jax.ShapeDtypeStruct((B, 16), jnp.float32),
        ),
        mesh=mesh,
        scratch_types=[
            pltpu.VMEM((GPE,), jnp.int32),
            pltpu.VMEM((GPW,), jnp.int32),
            pltpu.VMEM((GPE, D), jnp.bfloat16),
            pltpu.VMEM((GPW + 16,), jnp.float32),
            pltpu.VMEM((RPW, 16), jnp.float32),
            pltpu.SemaphoreType.DMA,
            pltpu.SemaphoreType.DMA,
            pltpu.SemaphoreType.DMA,
        ],
        compiler_params=pltpu.CompilerParams(use_tc_tiling_on_sc=False),
    )
    return k(emb_flat, fm_flat, gidx_flat, idx_flat)


VCH = 16384
NVCH = -(-V // VCH)
FG = 8
NG = -(-F // FG)


def _relayout_body(in_ref, out_ref):
    eye = jnp.eye(FG * D, dtype=jnp.float32)
    x = in_ref[...].reshape(FG * D, VCH)
    z = lax.dot_general(x, eye, (((0,), (0,)), ((), ())),
                        preferred_element_type=jnp.float32)
    out_ref[...] = z.astype(jnp.bfloat16)[None]


def _relayout(emb_t):
    out = pl.pallas_call(
        _relayout_body,
        grid=(NG, NVCH),
        in_specs=[pl.BlockSpec((FG, D, VCH), lambda g, c: (g, 0, c))],
        out_specs=pl.BlockSpec((1, VCH, FG * D), lambda g, c: (g, c, 0)),
        out_shape=jax.ShapeDtypeStruct((NG, V, FG * D), jnp.bfloat16),
    )(emb_t)
    return out.reshape(NG * V * FG, D)


FCH = 65536
NFCH = -(-F * V // FCH)


def _fm_body(in_ref, out_ref):
    out_ref[...] = in_ref[0]


def _fm_flatten(fm_t):
    return pl.pallas_call(
        _fm_body,
        grid=(NFCH,),
        in_specs=[pl.BlockSpec((1, FCH), lambda i: (0, i))],
        out_specs=pl.BlockSpec((FCH,), lambda i: (i,)),
        out_shape=jax.ShapeDtypeStruct((F * V,), jnp.float32),
    )(fm_t)


def _mlp_body(emb_ref, num_ref, fm_ref, w1_ref, s1_ref, b1_ref,
              w2_ref, s2_ref, b2_ref, w3_ref, s3_ref, b3_ref, w4_ref, out_ref):
    x = emb_ref[...][:, : F * D].astype(jnp.float32)
    sm = None
    sq = None
    for f in range(F):
        e = x[:, f * D : (f + 1) * D]
        sm = e if sm is None else sm + e
        sq = e * e if sq is None else sq + e * e
    fm2 = 0.5 * jnp.sum(sm * sm - sq, axis=1)
    h = jnp.dot(x, w1_ref[...][: F * D], preferred_element_type=jnp.float32)
    h = h + jnp.dot(num_ref[...], w1_ref[...][F * D : F * D + NUM],
                    preferred_element_type=jnp.float32)
    h = jnp.maximum(h * s1_ref[...] + b1_ref[...], 0.0)
    h = jnp.dot(h, w2_ref[...], preferred_element_type=jnp.float32)
    h = jnp.maximum(h * s2_ref[...] + b2_ref[...], 0.0)
    h = jnp.dot(h, w3_ref[...], preferred_element_type=jnp.float32)
    h = jnp.maximum(h * s3_ref[...] + b3_ref[...], 0.0)
    dnn = jnp.sum(h * w4_ref[...][None, :], axis=1)
    out_ref[...] = 1.0 / (1.0 + jnp.exp(-(dnn + fm2 + fm_ref[...][:, 0])))


def _mlp(emb2d, x_num, fmb, w1, s1, b1, w2, s2, b2, w3, s3, b3, w4row):
    blk = 512
    grid = B // blk

    def full(shape):
        return pl.BlockSpec(shape, lambda i: tuple(0 for _ in shape))

    return pl.pallas_call(
        _mlp_body,
        grid=(grid,),
        in_specs=[
            pl.BlockSpec((blk, SPR * D), lambda i: (i, 0)),
            pl.BlockSpec((blk, NUM), lambda i: (i, 0)),
            pl.BlockSpec((blk, 16), lambda i: (i, 0)),
            full((F * D + NUM, 1024)),
            full((1024,)),
            full((1024,)),
            full((1024, 512)),
            full((512,)),
            full((512,)),
            full((512, 256)),
            full((256,)),
            full((256,)),
            full((256,)),
        ],
        out_specs=pl.BlockSpec((blk,), lambda i: (i,)),
        out_shape=jax.ShapeDtypeStruct((B,), jnp.float32),
    )(emb2d, x_num, fmb, w1, s1, b1, w2, s2, b2, w3, s3, b3, w4row)


def kernel(x_cat, x_num, emb, fm_w, offsets, W1, b1, g1, beta1,
           W2, b2, g2, beta2, W3, b3, g3, beta3, W4, b4):
    fidx = jnp.arange(F, dtype=jnp.int32)
    grp = (fidx // FG) * (V * FG) + (fidx % FG)
    gidx = x_cat * FG + grp[None, :]
    padv = (jnp.arange(B, dtype=jnp.int32)[:, None] * (SPR - F)
            + jnp.arange(SPR - F, dtype=jnp.int32)[None, :]) % (NG * V * FG)
    gidx_flat = jnp.concatenate([gidx, padv], axis=1).reshape(-1)
    idx_flat = (x_cat + offsets[None, :]).reshape(-1)
    emb_flat = _relayout(jnp.transpose(emb, (0, 2, 1)))
    fm_flat = _fm_flatten(jnp.transpose(fm_w, (1, 0)))

    rows, fmb = _sc_gather_fm(emb_flat, fm_flat, gidx_flat, idx_flat)
    emb2d = rows.reshape(B, SPR * D)

    inv = 1.0 / jnp.sqrt(1.0 + 1e-5)
    s1, s2, s3 = g1 * inv, g2 * inv, g3 * inv
    b1f = b1 * s1 + beta1
    b2f = b2 * s2 + beta2
    b3f = b3 * s3 + beta3
    w4row = W4[:, 0]
    fmb = fmb + b4[0]

    return _mlp(emb2d, x_num, fmb, W1, s1, b1f, W2, s2, b2f, W3, s3, b3f, w4row)


# speedup vs baseline: 2.5692x; 2.5692x over previous
import jax
import jax.numpy as jnp
from jax import lax
from jax.experimental import pallas as pl
from jax.experimental.pallas import tpu as pltpu
from jax.experimental.pallas import tpu_sc as plsc

F = 26
V = 100000
D = 16
NUM = 13
B = 4096
NC = 2
NS = 16
NW = NC * NS
RPW = B // NW
GPW = RPW * F
SPR = 32
GPE = RPW * SPR
CHUNK = 128
NCHUNK = GPW // CHUNK
NCHE = GPE // CHUNK


def _sc_body(emb_hbm, fm_hbm, gidx_hbm, idx_hbm, rows_out, fm_out,
             gidx_v, idx_v, rows_v, fmv, fm_res, sem_e, sem_f, sem_o):
    wid = lax.axis_index("s") * NC + lax.axis_index("c")
    base = wid * GPW

    pltpu.sync_copy(gidx_hbm.at[pl.ds(wid * GPE, GPE)], gidx_v)
    pltpu.sync_copy(idx_hbm.at[pl.ds(base, GPW)], idx_v)

    def fire(j, c):
        gsl = gidx_v.at[pl.ds(j * CHUNK, CHUNK)]
        pltpu.async_copy(emb_hbm.at[gsl], rows_v.at[pl.ds(j * CHUNK, CHUNK)], sem_e)

        @pl.when(j < NCHUNK)
        def _():
            isl = idx_v.at[pl.ds(j * CHUNK, CHUNK)]
            pltpu.async_copy(fm_hbm.at[isl], fmv.at[pl.ds(j * CHUNK, CHUNK)], sem_f)

        return c

    lax.fori_loop(0, NCHE, fire, 0)
    pltpu.make_async_copy(emb_hbm.at[pl.ds(0, GPE)], rows_v, sem_e).wait()
    pltpu.make_async_copy(fm_hbm.at[pl.ds(0, GPW)], fmv.at[pl.ds(0, GPW)], sem_f).wait()

    out_copy = pltpu.make_async_copy(rows_v, rows_out.at[pl.ds(wid * GPE, GPE)], sem_o)
    out_copy.start()

    lanes = lax.iota(jnp.int32, 16)
    tail_mask = lanes < (F - 16)
    rots = [((lanes + k) & 15)[:, None] for k in (1, 2, 4, 8)]
    dnums = lax.GatherDimensionNumbers(
        offset_dims=(), collapsed_slice_dims=(0,), start_index_map=(0,))

    def lane_sum(x):
        for r in rots:
            x = x + lax.gather(x, r, dimension_numbers=dnums, slice_sizes=(1,),
                               mode=lax.GatherScatterMode.PROMISE_IN_BOUNDS)
        return x

    def per_row(i, c):
        base_g = i * SPR
        base_f = i * F
        s = rows_v[base_g]
        sq = s * s
        for f in range(1, F):
            e = rows_v[base_g + f]
            s = s + e
            sq = sq + e * e
        v0 = fmv[pl.ds(base_f, 16)]
        v1 = jnp.where(tail_mask, fmv[pl.ds(base_f + 16, 16)], 0.0)
        fm_res[i] = lane_sum(v0 + v1) + 0.5 * lane_sum(s * s - sq)
        return c

    lax.fori_loop(0, RPW, per_row, 0)

    pltpu.sync_copy(fm_res, fm_out.at[pl.ds(wid * RPW, RPW)])
    out_copy.wait()


def _sc_gather_fm(emb_flat, fm_flat, gidx_flat, idx_flat):
    mesh = plsc.VectorSubcoreMesh(core_axis_name="c", subcore_axis_name="s")
    k = pl.kernel(
        _sc_body,
        out_type=(
            jax.ShapeDtypeStruct((B * SPR, D), jnp.float32),
            jax.ShapeDtypeStruct((B, 16), jnp.float32),
        ),
        mesh=mesh,
        scratch_types=[
            pltpu.VMEM((GPE,), jnp.int32),
            pltpu.VMEM((GPW,), jnp.int32),
            pltpu.VMEM((GPE, D), jnp.float32),
            pltpu.VMEM((GPW + 16,), jnp.float32),
            pltpu.VMEM((RPW, 16), jnp.float32),
            pltpu.SemaphoreType.DMA,
            pltpu.SemaphoreType.DMA,
            pltpu.SemaphoreType.DMA,
        ],
        compiler_params=pltpu.CompilerParams(use_tc_tiling_on_sc=False),
    )
    return k(emb_flat, fm_flat, gidx_flat, idx_flat)


VCH = 16384
NVCH = -(-V // VCH)
FG = 8
NG = -(-F // FG)


def _relayout_body(in_ref, out_ref):
    eye = jnp.eye(FG * D, dtype=jnp.float32)
    x = in_ref[...].reshape(FG * D, VCH)
    out_ref[...] = lax.dot_general(
        x, eye, (((0,), (0,)), ((), ())),
        preferred_element_type=jnp.float32)[None]


def _relayout(emb_t):
    out = pl.pallas_call(
        _relayout_body,
        grid=(NG, NVCH),
        in_specs=[pl.BlockSpec((FG, D, VCH), lambda g, c: (g, 0, c))],
        out_specs=pl.BlockSpec((1, VCH, FG * D), lambda g, c: (g, c, 0)),
        out_shape=jax.ShapeDtypeStruct((NG, V, FG * D), jnp.float32),
    )(emb_t)
    return out.reshape(NG * V * FG, D)


FCH = 262144
NFCH = -(-F * V // FCH)


def _fm_body(in_ref, out_ref):
    out_ref[...] = in_ref[0]


def _fm_flatten(fm_t):
    return pl.pallas_call(
        _fm_body,
        grid=(NFCH,),
        in_specs=[pl.BlockSpec((1, FCH), lambda i: (0, i))],
        out_specs=pl.BlockSpec((FCH,), lambda i: (i,)),
        out_shape=jax.ShapeDtypeStruct((F * V,), jnp.float32),
    )(fm_t)


def _mlp_body(emb_ref, num_ref, fm_ref, w1_ref, s1_ref, b1_ref,
              w2_ref, s2_ref, b2_ref, w3_ref, s3_ref, b3_ref, w4_ref, out_ref):
    x = emb_ref[...][:, : F * D]
    h = jnp.dot(x, w1_ref[...][: F * D], preferred_element_type=jnp.float32)
    h = h + jnp.dot(num_ref[...], w1_ref[...][F * D : F * D + NUM],
                    preferred_element_type=jnp.float32)
    h = jnp.maximum(h * s1_ref[...] + b1_ref[...], 0.0)
    h = jnp.dot(h, w2_ref[...], preferred_element_type=jnp.float32)
    h = jnp.maximum(h * s2_ref[...] + b2_ref[...], 0.0)
    h = jnp.dot(h, w3_ref[...], preferred_element_type=jnp.float32)
    h = jnp.maximum(h * s3_ref[...] + b3_ref[...], 0.0)
    dnn = jnp.sum(h * w4_ref[...][None, :], axis=1)
    out_ref[...] = 1.0 / (1.0 + jnp.exp(-(dnn + fm_ref[...][:, 0])))


def _mlp(emb2d, x_num, fmb, w1, s1, b1, w2, s2, b2, w3, s3, b3, w4row):
    blk = 512
    grid = B // blk

    def full(shape):
        return pl.BlockSpec(shape, lambda i: tuple(0 for _ in shape))

    return pl.pallas_call(
        _mlp_body,
        grid=(grid,),
        in_specs=[
            pl.BlockSpec((blk, SPR * D), lambda i: (i, 0)),
            pl.BlockSpec((blk, NUM), lambda i: (i, 0)),
            pl.BlockSpec((blk, 16), lambda i: (i, 0)),
            full((F * D + NUM, 1024)),
            full((1024,)),
            full((1024,)),
            full((1024, 512)),
            full((512,)),
            full((512,)),
            full((512, 256)),
            full((256,)),
            full((256,)),
            full((256,)),
        ],
        out_specs=pl.BlockSpec((blk,), lambda i: (i,)),
        out_shape=jax.ShapeDtypeStruct((B,), jnp.float32),
    )(emb2d, x_num, fmb, w1, s1, b1, w2, s2, b2, w3, s3, b3, w4row)


def kernel(x_cat, x_num, emb, fm_w, offsets, W1, b1, g1, beta1,
           W2, b2, g2, beta2, W3, b3, g3, beta3, W4, b4):
    fidx = jnp.arange(F, dtype=jnp.int32)
    grp = (fidx // FG) * (V * FG) + (fidx % FG)
    gidx = x_cat * FG + grp[None, :]
    padv = (jnp.arange(B, dtype=jnp.int32)[:, None] * (SPR - F)
            + jnp.arange(SPR - F, dtype=jnp.int32)[None, :]) % (NG * V * FG)
    gidx_flat = jnp.concatenate([gidx, padv], axis=1).reshape(-1)
    idx_flat = (x_cat + offsets[None, :]).reshape(-1)
    emb_flat = _relayout(jnp.transpose(emb, (0, 2, 1)))
    fm_flat = _fm_flatten(jnp.transpose(fm_w, (1, 0)))

    rows, fmb = _sc_gather_fm(emb_flat, fm_flat, gidx_flat, idx_flat)
    emb2d = rows.reshape(B, SPR * D)

    inv = 1.0 / jnp.sqrt(1.0 + 1e-5)
    s1, s2, s3 = g1 * inv, g2 * inv, g3 * inv
    b1f = b1 * s1 + beta1
    b2f = b2 * s2 + beta2
    b3f = b3 * s3 + beta3
    w4row = W4[:, 0]
    fmb = fmb + b4[0]

    return _mlp(emb2d, x_num, fmb, W1, s1, b1f, W2, s2, b2f, W3, s3, b3f, w4row)
